# Initial kernel scaffold; baseline (speedup 1.0000x reference)
import jax
import jax.numpy as jnp
from jax.experimental import pallas as pl


def kernel(flower_label, color_label, flower_table, color_table, W, b):
    raise NotImplementedError("write your pallas kernel here")


# speedup vs baseline: 3.8811x; 3.8811x over previous
import functools

import jax
import jax.numpy as jnp
from jax import lax
from jax.experimental import pallas as pl
from jax.experimental.pallas import tpu as pltpu
from jax.experimental.pallas import tpu_sc as plsc


def _fuse_body(ft_ref, ct_ref, w_ref, b_ref, out_ref):
    c = ft_ref.shape[1]
    w1 = w_ref[:, :c]
    w2 = w_ref[:, c:]
    f = lax.dot_general(ft_ref[...], w1, (((1,), (1,)), ((), ())),
                        preferred_element_type=jnp.float32)
    g = lax.dot_general(ct_ref[...], w2, (((1,), (1,)), ((), ())),
                        preferred_element_type=jnp.float32) + b_ref[...]
    out_ref[...] = f[:, None, :] + g[None, :, :]


def _fused_table(flower_table, color_table, W, b):
    nf, c = flower_table.shape
    nc = color_table.shape[0]
    out3 = pl.pallas_call(
        _fuse_body,
        out_shape=jax.ShapeDtypeStruct((nf, nc, c), jnp.float32),
    )(flower_table, color_table, W, b.reshape(1, c))
    return out3.reshape(nf * nc, c)


def _sc_lookup(table, flower_label, color_label, ncolor):
    batch = flower_label.shape[0]
    c = table.shape[1]
    info = plsc.get_sparse_core_info()
    nw = info.num_cores * info.num_subcores
    bpw = batch // nw
    ch = 128
    nch = bpw // ch
    lanes = info.num_lanes

    mesh = plsc.VectorSubcoreMesh(core_axis_name="c", subcore_axis_name="s")

    @functools.partial(
        pl.kernel,
        mesh=mesh,
        out_type=jax.ShapeDtypeStruct((batch, c), jnp.float32),
        scratch_types=[
            pltpu.VMEM((bpw,), jnp.int32),
            pltpu.VMEM((bpw,), jnp.int32),
            pltpu.VMEM((bpw,), jnp.int32),
            pltpu.VMEM((ch, c), jnp.float32),
            pltpu.VMEM((ch, c), jnp.float32),
            pltpu.SemaphoreType.DMA,
            pltpu.SemaphoreType.DMA,
        ],
    )
    def k(fl_hbm, cl_hbm, tab_hbm, out_hbm, fidx, cidx, idx, buf0, buf1,
          sem0, sem1):
        wid = lax.axis_index("s") * info.num_cores + lax.axis_index("c")
        base = wid * bpw
        pltpu.sync_copy(fl_hbm.at[pl.ds(base, bpw)], fidx)
        pltpu.sync_copy(cl_hbm.at[pl.ds(base, bpw)], cidx)
        for i in range(bpw // lanes):
            s = pl.ds(i * lanes, lanes)
            idx[s] = fidx[s] * ncolor + cidx[s]
        bufs = (buf0, buf1)
        sems = (sem0, sem1)
        cps = [None] * nch
        for j in range(min(2, nch)):
            cps[j] = pltpu.async_copy(
                tab_hbm.at[idx.at[pl.ds(j * ch, ch)]], bufs[j % 2], sems[j % 2])
        for j in range(nch):
            cps[j].wait()
            pltpu.sync_copy(bufs[j % 2], out_hbm.at[pl.ds(base + j * ch, ch)])
            nxt = j + 2
            if nxt < nch:
                cps[nxt] = pltpu.async_copy(
                    tab_hbm.at[idx.at[pl.ds(nxt * ch, ch)]],
                    bufs[nxt % 2], sems[nxt % 2])

    return k(flower_label, color_label, table)


def kernel(flower_label, color_label, flower_table, color_table, W, b):
    tab = _fused_table(flower_table, color_table, W, b)
    return _sc_lookup(tab,
                      flower_label.astype(jnp.int32),
                      color_label.astype(jnp.int32),
                      color_table.shape[0])


# speedup vs baseline: 4.1080x; 1.0584x over previous
import functools

import jax
import jax.numpy as jnp
from jax import lax
from jax.experimental import pallas as pl
from jax.experimental.pallas import tpu as pltpu
from jax.experimental.pallas import tpu_sc as plsc


def _fuse_body(fl_ref, cl_ref, ft_ref, ct_ref, w_ref, b_ref, out_ref,
               idx_ref):
    nf, c = ft_ref.shape
    ncol = ct_ref.shape[0]
    rows = nf * ncol
    w1 = w_ref[:, :c]
    w2 = w_ref[:, c:]
    f = lax.dot_general(ft_ref[...], w1, (((1,), (1,)), ((), ())),
                        preferred_element_type=jnp.float32)
    g = lax.dot_general(ct_ref[...], w2, (((1,), (1,)), ((), ())),
                        preferred_element_type=jnp.float32) + b_ref[...]
    rf = lax.broadcasted_iota(jnp.int32, (rows, nf), 0)
    cf = lax.broadcasted_iota(jnp.int32, (rows, nf), 1)
    ohf = (rf // ncol == cf).astype(jnp.float32)
    rc = lax.broadcasted_iota(jnp.int32, (rows, ncol), 0)
    cc = lax.broadcasted_iota(jnp.int32, (rows, ncol), 1)
    ohc = (rc % ncol == cc).astype(jnp.float32)
    out_ref[...] = (
        lax.dot_general(ohf, f, (((1,), (0,)), ((), ())),
                        precision=lax.Precision.HIGHEST,
                        preferred_element_type=jnp.float32)
        + lax.dot_general(ohc, g, (((1,), (0,)), ((), ())),
                          precision=lax.Precision.HIGHEST,
                          preferred_element_type=jnp.float32))
    idx_ref[...] = fl_ref[...] * ncol + cl_ref[...]


def _fused_table(flower_label, color_label, flower_table, color_table, W, b):
    nf, c = flower_table.shape
    nc = color_table.shape[0]
    batch = flower_label.shape[0]
    return pl.pallas_call(
        _fuse_body,
        out_shape=(jax.ShapeDtypeStruct((nf * nc, c), jnp.float32),
                   jax.ShapeDtypeStruct((batch,), jnp.int32)),
    )(flower_label, color_label, flower_table, color_table, W,
      b.reshape(1, c))


def _sc_lookup(table, idx):
    batch = idx.shape[0]
    c = table.shape[1]
    info = plsc.get_sparse_core_info()
    nw = info.num_cores * info.num_subcores
    bpw = batch // nw
    ch = 64
    nch = bpw // ch
    nbuf = 7

    mesh = plsc.VectorSubcoreMesh(core_axis_name="c", subcore_axis_name="s")

    @functools.partial(
        pl.kernel,
        mesh=mesh,
        out_type=jax.ShapeDtypeStruct((batch, c), jnp.float32),
        scratch_types=[
            pltpu.VMEM((bpw,), jnp.int32),
        ] + [pltpu.VMEM((ch, c), jnp.float32)] * nbuf
          + [pltpu.SemaphoreType.DMA] * (2 * nbuf),
    )
    def k(idx_hbm, tab_hbm, out_hbm, idx2, *rest):
        bufs = rest[:nbuf]
        gsem = rest[nbuf:2 * nbuf]
        wsem = rest[2 * nbuf:]
        wid = lax.axis_index("s") * info.num_cores + lax.axis_index("c")
        base = wid * bpw
        pltpu.sync_copy(idx_hbm.at[pl.ds(base, bpw)], idx2)
        gcp = [None] * nch
        wcp = [None] * nch
        for j in range(min(nbuf, nch)):
            gcp[j] = pltpu.async_copy(
                tab_hbm.at[idx2.at[pl.ds(j * ch, ch)]], bufs[j % nbuf],
                gsem[j % nbuf])
        for j in range(nch):
            gcp[j].wait()
            wcp[j] = pltpu.async_copy(
                bufs[j % nbuf], out_hbm.at[pl.ds(base + j * ch, ch)],
                wsem[j % nbuf])
            nxt = j + nbuf
            if nxt < nch:
                wcp[j].wait()
                wcp[j] = None
                gcp[nxt] = pltpu.async_copy(
                    tab_hbm.at[idx2.at[pl.ds(nxt * ch, ch)]],
                    bufs[nxt % nbuf], gsem[nxt % nbuf])
        for j in range(nch):
            if wcp[j] is not None:
                wcp[j].wait()

    return k(idx, table)


def kernel(flower_label, color_label, flower_table, color_table, W, b):
    tab, idx = _fused_table(flower_label.astype(jnp.int32),
                            color_label.astype(jnp.int32),
                            flower_table, color_table, W, b)
    return _sc_lookup(tab, idx)
